# Initial kernel scaffold; baseline (speedup 1.0000x reference)
import jax
import jax.numpy as jnp
from jax.experimental import pallas as pl


def kernel(features, edge_index, edge_attr, Wb, bb, W1, b1, W2, b2, g1, be1, g2, be2):
    raise NotImplementedError("write your pallas kernel here")


# speedup vs baseline: 3.3178x; 3.3178x over previous
import functools

import jax
import jax.numpy as jnp
from jax import lax
from jax.experimental import pallas as pl
from jax.experimental.pallas import tpu as pltpu
from jax.experimental.pallas import tpu_sc as plsc

N = 10000
E = 320000
H = 128
DE = 16

NC = 2
NS = 16
CH = 80
EPT = E // NS
NCH1 = EPT // CH
ACH = 128
NAUX = E // ACH
AUXI = (NAUX + NS - 1) // NS
EAR = E * DE // H
NP = 10240
ROWS = NP // NS

_mesh = plsc.VectorSubcoreMesh(core_axis_name="c", subcore_axis_name="s")


@functools.partial(
    pl.kernel,
    out_type=jax.ShapeDtypeStruct((NC * NP, H), jnp.float32),
    mesh=_mesh,
    scratch_types=[
        pltpu.VMEM((CH,), jnp.int32),
        pltpu.VMEM((CH,), jnp.int32),
        pltpu.VMEM((ACH,), jnp.int32),
        pltpu.VMEM((CH, H), jnp.float32),
        pltpu.VMEM((ACH // 8, H), jnp.float32),
        pltpu.VMEM((ACH, H), jnp.float32),
        pltpu.VMEM_SHARED((NP, H), jnp.float32),
        pltpu.SemaphoreType.DMA,
    ],
)
def _sc_segment_sums(feat_hbm, src_hbm, dst_hbm, ear_hbm, zh_hbm,
                     out_hbm,
                     sidx, didx, didxa, rows_v, eablk_v, pay_v,
                     acc, sem):
    cid = lax.axis_index("c")
    sid = lax.axis_index("s")

    r0 = sid * ROWS
    pltpu.sync_copy(zh_hbm, rows_v)

    def zero_blk(j, carry):
        pltpu.sync_copy(rows_v, acc.at[pl.ds(r0 + j * CH, CH)])
        return carry

    lax.fori_loop(0, ROWS // CH, zero_blk, 0)
    plsc.subcore_barrier()

    @pl.when(cid == 0)
    def _():
        pltpu.sync_copy(zh_hbm, pay_v.at[pl.ds(0, CH)])
        pltpu.sync_copy(zh_hbm, pay_v.at[pl.ds(ACH - CH, CH)])

        def pay_init(j, carry):
            one_hot0 = jnp.maximum(1 - lax.iota(jnp.int32, 16), 0).astype(jnp.float32)
            pay_v[j, pl.ds(DE, 16)] = one_hot0
            return carry

        lax.fori_loop(0, ACH, pay_init, 0)

        def chunk_aux(i, carry):
            g = sid + NS * i

            @pl.when(g < NAUX)
            def _():
                pltpu.sync_copy(dst_hbm.at[pl.ds(g * ACH, ACH)], didxa)
                pltpu.sync_copy(ear_hbm.at[pl.ds(g * (ACH // 8), ACH // 8)],
                                eablk_v)

                def unpack_row(r, carry2):
                    for k in range(8):
                        pay_v[r * 8 + k, pl.ds(0, DE)] = eablk_v[r, pl.ds(k * DE, DE)]
                    return carry2

                lax.fori_loop(0, ACH // 8, unpack_row, 0)
                pltpu.sync_copy(pay_v, acc.at[didxa], add=True)

            return carry

        lax.fori_loop(0, AUXI, chunk_aux, 0)

    @pl.when(cid == 1)
    def _():
        ebase = sid * EPT

        def chunk(i, carry):
            base = ebase + i * CH
            pltpu.sync_copy(src_hbm.at[pl.ds(base, CH)], sidx)
            pltpu.sync_copy(dst_hbm.at[pl.ds(base, CH)], didx)
            pltpu.async_copy(feat_hbm.at[sidx], rows_v, sem).wait()
            pltpu.sync_copy(rows_v, acc.at[didx], add=True)
            return carry

        lax.fori_loop(0, NCH1, chunk, 0)

    plsc.subcore_barrier()

    def wb_blk(j, carry):
        rb = r0 + j * CH
        pltpu.sync_copy(acc.at[pl.ds(rb, CH)], rows_v)
        pltpu.sync_copy(rows_v, out_hbm.at[pl.ds(cid * NP + rb, CH)])
        return carry

    lax.fori_loop(0, ROWS // CH, wb_blk, 0)


def _tc_mlp_body(sc_ref, feat_ref, Wb_ref, bb_ref,
                 W1_ref, b1_ref, W2_ref, b2_ref, g1_ref, be1_ref,
                 g2_ref, be2_ref, out_ref):
    f32 = jnp.float32
    ea = sc_ref[:N, 0:DE]
    deg = sc_ref[:N, DE:DE + 1]
    h1 = sc_ref[NP:NP + N, :]
    h2 = jnp.dot(ea, Wb_ref[...], preferred_element_type=f32) + deg * bb_ref[...]
    t = jnp.dot(h1, W1_ref[...], preferred_element_type=f32) + b1_ref[...] + h2
    t = jnp.maximum(t, 0.0)
    mu = jnp.mean(t, axis=0, keepdims=True)
    var = jnp.mean(t * t, axis=0, keepdims=True) - mu * mu
    t = (t - mu) * lax.rsqrt(var + 1e-5) * g1_ref[...] + be1_ref[...]
    t2 = jnp.dot(t, W2_ref[...], preferred_element_type=f32) + b2_ref[...] + feat_ref[...]
    t2 = jnp.maximum(t2, 0.0)
    mu2 = jnp.mean(t2, axis=0, keepdims=True)
    var2 = jnp.mean(t2 * t2, axis=0, keepdims=True) - mu2 * mu2
    out_ref[...] = (t2 - mu2) * lax.rsqrt(var2 + 1e-5) * g2_ref[...] + be2_ref[...]


_tc_mlp = pl.pallas_call(
    _tc_mlp_body,
    out_shape=jax.ShapeDtypeStruct((N, H), jnp.float32),
)


def kernel(features, edge_index, edge_attr, Wb, bb, W1, b1, W2, b2, g1, be1, g2, be2):
    src = edge_index[0]
    dst = edge_index[1]
    ea_r = edge_attr.reshape(EAR, H)
    zeros_h = jnp.zeros((CH, H), jnp.float32)
    scout = _sc_segment_sums(features, src, dst, ea_r, zeros_h)
    return _tc_mlp(scout, features,
                   Wb, bb.reshape(1, H), W1, b1.reshape(1, H),
                   W2, b2.reshape(1, H), g1.reshape(1, H), be1.reshape(1, H),
                   g2.reshape(1, H), be2.reshape(1, H))


# speedup vs baseline: 3.8565x; 1.1623x over previous
import functools

import jax
import jax.numpy as jnp
from jax import lax
from jax.experimental import pallas as pl
from jax.experimental.pallas import tpu as pltpu
from jax.experimental.pallas import tpu_sc as plsc

N = 10000
E = 320000
H = 128
DE = 16

NC = 2
NS = 16
CH = 80
EPT = E // NS
NCH1 = EPT // CH
ACH = 128
NAUX = E // ACH
AUXI = (NAUX + NS - 1) // NS
EAR = E * DE // H
NP = 10240
ROWS = NP // NS

_mesh = plsc.VectorSubcoreMesh(core_axis_name="c", subcore_axis_name="s")


@functools.partial(
    pl.kernel,
    out_type=jax.ShapeDtypeStruct((NC * NP, H), jnp.float32),
    mesh=_mesh,
    scratch_types=[
        pltpu.VMEM((2, CH), jnp.int32),
        pltpu.VMEM((2, CH), jnp.int32),
        pltpu.VMEM((2, ACH), jnp.int32),
        pltpu.VMEM((CH, H), jnp.float32),
        pltpu.VMEM((CH, H), jnp.float32),
        pltpu.VMEM((ACH // 8, H), jnp.float32),
        pltpu.VMEM((ACH // 8, H), jnp.float32),
        pltpu.VMEM((ACH, H), jnp.float32),
        pltpu.VMEM_SHARED((NP, H), jnp.float32),
        pltpu.SemaphoreType.DMA,
        pltpu.SemaphoreType.DMA,
        pltpu.SemaphoreType.DMA,
        pltpu.SemaphoreType.DMA,
    ],
)
def _sc_segment_sums(feat_hbm, src_hbm, dst_hbm, ear_hbm, zh_hbm,
                     out_hbm,
                     sidx2, didx2, didxa2, rows_a, rows_b, eablk_a, eablk_b,
                     pay_v, acc, gsem_a, gsem_b, esem_a, esem_b):
    cid = lax.axis_index("c")
    sid = lax.axis_index("s")

    r0 = sid * ROWS
    pltpu.sync_copy(zh_hbm, rows_a)

    def zero_blk(j, carry):
        pltpu.sync_copy(rows_a, acc.at[pl.ds(r0 + j * CH, CH)])
        return carry

    lax.fori_loop(0, ROWS // CH, zero_blk, 0)
    plsc.subcore_barrier()

    @pl.when(cid == 0)
    def _():
        pltpu.sync_copy(zh_hbm, pay_v.at[pl.ds(0, CH)])
        pltpu.sync_copy(zh_hbm, pay_v.at[pl.ds(ACH - CH, CH)])

        def pay_init(j, carry):
            one_hot0 = jnp.maximum(1 - lax.iota(jnp.int32, 16), 0).astype(jnp.float32)
            pay_v[j, pl.ds(DE, 16)] = one_hot0
            return carry

        lax.fori_loop(0, ACH, pay_init, 0)

        eabufs = (eablk_a, eablk_b)
        esems = (esem_a, esem_b)

        def aux_load(i, b):
            g = sid + NS * i
            pltpu.sync_copy(dst_hbm.at[pl.ds(g * ACH, ACH)], didxa2.at[b])
            pltpu.async_copy(ear_hbm.at[pl.ds(g * (ACH // 8), ACH // 8)],
                             eabufs[b], esems[b])

        @pl.when(sid < NAUX)
        def _():
            aux_load(0, 0)

        def chunk_aux(i, carry):
            for b in range(2):
                c = 2 * i + b

                @pl.when(sid + NS * c < NAUX)
                def _():
                    pltpu.make_async_copy(
                        ear_hbm.at[pl.ds(0, ACH // 8)], eabufs[b],
                        esems[b]).wait()
                    ob = 1 - b

                    @pl.when(sid + NS * (c + 1) < NAUX)
                    def _():
                        aux_load(c + 1, ob)

                    def unpack_row(r, carry2):
                        for k in range(8):
                            pay_v[r * 8 + k, pl.ds(0, DE)] = eabufs[b][r, pl.ds(k * DE, DE)]
                        return carry2

                    lax.fori_loop(0, ACH // 8, unpack_row, 0)
                    pltpu.sync_copy(pay_v, acc.at[didxa2.at[b]], add=True)

            return carry

        lax.fori_loop(0, (AUXI + 1) // 2, chunk_aux, 0)

    @pl.when(cid == 1)
    def _():
        ebase = sid * EPT
        rowbufs = (rows_a, rows_b)
        gsems = (gsem_a, gsem_b)

        def idx_load(c, b):
            base = ebase + c * CH
            pltpu.sync_copy(src_hbm.at[pl.ds(base, CH)], sidx2.at[b])
            pltpu.sync_copy(dst_hbm.at[pl.ds(base, CH)], didx2.at[b])

        idx_load(0, 0)
        pltpu.async_copy(feat_hbm.at[sidx2.at[0]], rowbufs[0], gsems[0])

        def chunk(i, carry):
            for b in range(2):
                c = 2 * i + b
                ob = 1 - b
                pltpu.make_async_copy(feat_hbm.at[sidx2.at[b]], rowbufs[b],
                                      gsems[b]).wait()

                @pl.when(c + 1 < NCH1)
                def _():
                    idx_load(c + 1, ob)
                    pltpu.async_copy(feat_hbm.at[sidx2.at[ob]], rowbufs[ob],
                                     gsems[ob])

                pltpu.sync_copy(rowbufs[b], acc.at[didx2.at[b]], add=True)
            return carry

        lax.fori_loop(0, NCH1 // 2, chunk, 0)

    plsc.subcore_barrier()

    def wb_blk(j, carry):
        rb = r0 + j * CH
        pltpu.sync_copy(acc.at[pl.ds(rb, CH)], rows_a)
        pltpu.sync_copy(rows_a, out_hbm.at[pl.ds(cid * NP + rb, CH)])
        return carry

    lax.fori_loop(0, ROWS // CH, wb_blk, 0)


def _tc_mlp_body(sc_ref, feat_ref, Wb_ref, bb_ref,
                 W1_ref, b1_ref, W2_ref, b2_ref, g1_ref, be1_ref,
                 g2_ref, be2_ref, out_ref):
    f32 = jnp.float32
    ea = sc_ref[:N, 0:DE]
    deg = sc_ref[:N, DE:DE + 1]
    h1 = sc_ref[NP:NP + N, :]
    h2 = jnp.dot(ea, Wb_ref[...], preferred_element_type=f32) + deg * bb_ref[...]
    t = jnp.dot(h1, W1_ref[...], preferred_element_type=f32) + b1_ref[...] + h2
    t = jnp.maximum(t, 0.0)
    mu = jnp.mean(t, axis=0, keepdims=True)
    var = jnp.mean(t * t, axis=0, keepdims=True) - mu * mu
    t = (t - mu) * lax.rsqrt(var + 1e-5) * g1_ref[...] + be1_ref[...]
    t2 = jnp.dot(t, W2_ref[...], preferred_element_type=f32) + b2_ref[...] + feat_ref[...]
    t2 = jnp.maximum(t2, 0.0)
    mu2 = jnp.mean(t2, axis=0, keepdims=True)
    var2 = jnp.mean(t2 * t2, axis=0, keepdims=True) - mu2 * mu2
    out_ref[...] = (t2 - mu2) * lax.rsqrt(var2 + 1e-5) * g2_ref[...] + be2_ref[...]


_tc_mlp = pl.pallas_call(
    _tc_mlp_body,
    out_shape=jax.ShapeDtypeStruct((N, H), jnp.float32),
)


def kernel(features, edge_index, edge_attr, Wb, bb, W1, b1, W2, b2, g1, be1, g2, be2):
    src = edge_index[0]
    dst = edge_index[1]
    ea_r = edge_attr.reshape(EAR, H)
    zeros_h = jnp.zeros((CH, H), jnp.float32)
    scout = _sc_segment_sums(features, src, dst, ea_r, zeros_h)
    return _tc_mlp(scout, features,
                   Wb, bb.reshape(1, H), W1, b1.reshape(1, H),
                   W2, b2.reshape(1, H), g1.reshape(1, H), be1.reshape(1, H),
                   g2.reshape(1, H), be2.reshape(1, H))


# speedup vs baseline: 5.8912x; 1.5276x over previous
import functools

import jax
import jax.numpy as jnp
from jax import lax
from jax.experimental import pallas as pl
from jax.experimental.pallas import tpu as pltpu
from jax.experimental.pallas import tpu_sc as plsc

N = 10000
E = 320000
H = 128
DE = 16

NC = 2
NS = 16
CH = 80
EPT = E // NS
NCH1 = EPT // CH
ACH = 128
NAUX = E // ACH
AUXI = (NAUX + NS - 1) // NS
EAR = E * DE // H
NP = 10240
ROWS = NP // NS

_mesh = plsc.VectorSubcoreMesh(core_axis_name="c", subcore_axis_name="s")


@functools.partial(
    pl.kernel,
    out_type=jax.ShapeDtypeStruct((NC * NP, H), jnp.float32),
    mesh=_mesh,
    scratch_types=[
        pltpu.VMEM((4, CH), jnp.int32),
        pltpu.VMEM((4, CH), jnp.int32),
        pltpu.VMEM((2, ACH), jnp.int32),
        pltpu.VMEM((CH, H), jnp.float32),
        pltpu.VMEM((CH, H), jnp.float32),
        pltpu.VMEM((ACH // 8, H), jnp.float32),
        pltpu.VMEM((ACH // 8, H), jnp.float32),
        pltpu.VMEM((ACH, H), jnp.float32),
        pltpu.VMEM_SHARED((NP, H), jnp.float32),
        pltpu.SemaphoreType.DMA,
        pltpu.SemaphoreType.DMA,
        pltpu.SemaphoreType.DMA,
        pltpu.SemaphoreType.DMA,
        pltpu.SemaphoreType.DMA,
        pltpu.SemaphoreType.DMA,
        pltpu.SemaphoreType.DMA,
        pltpu.SemaphoreType.DMA,
    ],
)
def _sc_segment_sums(feat_hbm, src_hbm, dst_hbm, ear_hbm, zh_hbm,
                     out_hbm,
                     sidx2, didx2, didxa2, rows_a, rows_b,
                     eablk_a, eablk_b, pay_v, acc,
                     gsem_a, gsem_b, isem_a, isem_b, isem_c, isem_d,
                     esem_a, esem_b):
    cid = lax.axis_index("c")
    sid = lax.axis_index("s")

    r0 = sid * ROWS
    pltpu.sync_copy(zh_hbm, rows_a)

    def zero_blk(j, carry):
        pltpu.sync_copy(rows_a, acc.at[pl.ds(r0 + j * CH, CH)])
        return carry

    lax.fori_loop(0, ROWS // CH, zero_blk, 0)
    plsc.subcore_barrier()

    @pl.when(cid == 0)
    def _():
        pltpu.sync_copy(zh_hbm, pay_v.at[pl.ds(0, CH)])
        pltpu.sync_copy(zh_hbm, pay_v.at[pl.ds(ACH - CH, CH)])

        def pay_init(j, carry):
            one_hot0 = jnp.maximum(1 - lax.iota(jnp.int32, 16), 0).astype(jnp.float32)
            pay_v[j, pl.ds(DE, 16)] = one_hot0
            return carry

        lax.fori_loop(0, ACH, pay_init, 0)

        eabufs = (eablk_a, eablk_b)
        esems = (esem_a, esem_b)

        def aux_load(i, b):
            g = sid + NS * i
            pltpu.sync_copy(dst_hbm.at[pl.ds(g * ACH, ACH)], didxa2.at[b])
            pltpu.async_copy(ear_hbm.at[pl.ds(g * (ACH // 8), ACH // 8)],
                             eabufs[b], esems[b])

        @pl.when(sid < NAUX)
        def _():
            aux_load(0, 0)

        def chunk_aux(i, carry):
            for b in range(2):
                c = 2 * i + b

                @pl.when(sid + NS * c < NAUX)
                def _():
                    pltpu.make_async_copy(
                        ear_hbm.at[pl.ds(0, ACH // 8)], eabufs[b],
                        esems[b]).wait()
                    ob = 1 - b

                    @pl.when(sid + NS * (c + 1) < NAUX)
                    def _():
                        aux_load(c + 1, ob)

                    def unpack_row(r, carry2):
                        for k in range(8):
                            pay_v[r * 8 + k, pl.ds(0, DE)] = eabufs[b][r, pl.ds(k * DE, DE)]
                        return carry2

                    lax.fori_loop(0, ACH // 8, unpack_row, 0)
                    pltpu.sync_copy(pay_v, acc.at[didxa2.at[b]], add=True)

            return carry

        lax.fori_loop(0, (AUXI + 1) // 2, chunk_aux, 0)

    @pl.when(cid == 1)
    def _():
        ebase = sid * EPT
        rowbufs = (rows_a, rows_b)
        gsems = (gsem_a, gsem_b)
        isems = (isem_a, isem_b, isem_c, isem_d)

        def idx_start(c, s):
            base = ebase + c * CH
            pltpu.async_copy(src_hbm.at[pl.ds(base, CH)], sidx2.at[s], isems[s])
            pltpu.async_copy(dst_hbm.at[pl.ds(base, CH)], didx2.at[s], isems[s])

        def idx_wait(s):
            pltpu.make_async_copy(src_hbm.at[pl.ds(0, CH)], sidx2.at[s],
                                  isems[s]).wait()
            pltpu.make_async_copy(dst_hbm.at[pl.ds(0, CH)], didx2.at[s],
                                  isems[s]).wait()

        idx_start(0, 0)
        idx_start(1, 1)
        idx_wait(0)
        pltpu.async_copy(feat_hbm.at[sidx2.at[0]], rowbufs[0], gsems[0])

        def chunk(i, carry):
            for u in range(4):
                c = 4 * i + u
                s = u
                b = u % 2

                @pl.when(c < NCH1)
                def _():
                    s1 = (u + 1) % 4
                    s2 = (u + 2) % 4
                    b1 = (u + 1) % 2

                    @pl.when(c + 2 < NCH1)
                    def _():
                        idx_start(c + 2, s2)

                    @pl.when(c + 1 < NCH1)
                    def _():
                        idx_wait(s1)
                        pltpu.async_copy(feat_hbm.at[sidx2.at[s1]],
                                         rowbufs[b1], gsems[b1])

                    pltpu.make_async_copy(feat_hbm.at[sidx2.at[s]],
                                          rowbufs[b], gsems[b]).wait()
                    pltpu.sync_copy(rowbufs[b], acc.at[didx2.at[s]], add=True)

            return carry

        lax.fori_loop(0, (NCH1 + 3) // 4, chunk, 0)

    plsc.subcore_barrier()

    def wb_blk(j, carry):
        rb = r0 + j * CH
        pltpu.sync_copy(acc.at[pl.ds(rb, CH)], rows_a)
        pltpu.sync_copy(rows_a, out_hbm.at[pl.ds(cid * NP + rb, CH)])
        return carry

    lax.fori_loop(0, ROWS // CH, wb_blk, 0)


def _tc_mlp_body(sc_ref, feat_ref, Wb_ref, bb_ref,
                 W1_ref, b1_ref, W2_ref, b2_ref, g1_ref, be1_ref,
                 g2_ref, be2_ref, out_ref):
    f32 = jnp.float32
    ea = sc_ref[:N, 0:DE]
    deg = sc_ref[:N, DE:DE + 1]
    h1 = sc_ref[NP:NP + N, :]
    h2 = jnp.dot(ea, Wb_ref[...], preferred_element_type=f32) + deg * bb_ref[...]
    t = jnp.dot(h1, W1_ref[...], preferred_element_type=f32) + b1_ref[...] + h2
    t = jnp.maximum(t, 0.0)
    mu = jnp.mean(t, axis=0, keepdims=True)
    var = jnp.mean(t * t, axis=0, keepdims=True) - mu * mu
    t = (t - mu) * lax.rsqrt(var + 1e-5) * g1_ref[...] + be1_ref[...]
    t2 = jnp.dot(t, W2_ref[...], preferred_element_type=f32) + b2_ref[...] + feat_ref[...]
    t2 = jnp.maximum(t2, 0.0)
    mu2 = jnp.mean(t2, axis=0, keepdims=True)
    var2 = jnp.mean(t2 * t2, axis=0, keepdims=True) - mu2 * mu2
    out_ref[...] = (t2 - mu2) * lax.rsqrt(var2 + 1e-5) * g2_ref[...] + be2_ref[...]


_tc_mlp = pl.pallas_call(
    _tc_mlp_body,
    out_shape=jax.ShapeDtypeStruct((N, H), jnp.float32),
)


def kernel(features, edge_index, edge_attr, Wb, bb, W1, b1, W2, b2, g1, be1, g2, be2):
    src = edge_index[0]
    dst = edge_index[1]
    ea_r = edge_attr.reshape(EAR, H)
    zeros_h = jnp.zeros((CH, H), jnp.float32)
    scout = _sc_segment_sums(features, src, dst, ea_r, zeros_h)
    return _tc_mlp(scout, features,
                   Wb, bb.reshape(1, H), W1, b1.reshape(1, H),
                   W2, b2.reshape(1, H), g1.reshape(1, H), be1.reshape(1, H),
                   g2.reshape(1, H), be2.reshape(1, H))
